# Initial kernel scaffold; baseline (speedup 1.0000x reference)
import jax
import jax.numpy as jnp
from jax.experimental import pallas as pl


def kernel(hidden_states, Wq, Wk, Wv, Wfc, bfc, gamma, beta):
    raise NotImplementedError("write your pallas kernel here")


# speedup vs baseline: 3.5866x; 3.5866x over previous
import math

import numpy as np
import jax
import jax.numpy as jnp
from jax.experimental import pallas as pl
from jax.experimental.pallas import tpu as pltpu

L = 2048
DM = 768
H = 12
DK = 64
U = min(5 * int(np.ceil(np.log(L))), L)
EPS = 1e-6
NEG = float(np.float32(-3.0e38))


def _sample_counts_T() -> np.ndarray:
    idx = np.asarray(jax.random.randint(jax.random.key(42), (L, U), 0, L))
    cnt = np.zeros((L, L), np.int8)
    np.add.at(cnt, (np.arange(L)[:, None], idx), 1)
    return np.ascontiguousarray(cnt.T)


_CNT_T = _sample_counts_T()


def _attn_head_kernel(x_ref, wq_ref, wk_ref, wv_ref, cnt_ref, ctx_ref,
                      q_scr, idx_scr):
    x = x_ref[...]
    q = jnp.dot(x, wq_ref[0], preferred_element_type=jnp.float32)
    q = q * (1.0 / math.sqrt(DK))
    k = jnp.dot(x, wk_ref[0], preferred_element_type=jnp.float32)
    v = jnp.dot(x, wv_ref[0], preferred_element_type=jnp.float32)
    q_scr[...] = q

    KB = 512
    runmax = jnp.full((1, L), NEG, jnp.float32)
    runsum = jnp.zeros((1, L), jnp.float32)
    for b in range(L // KB):
        kb = k[b * KB:(b + 1) * KB, :]
        s = jax.lax.dot_general(kb, q, (((1,), (1,)), ((), ())),
                                preferred_element_type=jnp.float32)
        cnt = cnt_ref[b * KB:(b + 1) * KB, :].astype(jnp.float32)
        runmax = jnp.maximum(
            runmax, jnp.max(jnp.where(cnt > 0, s, NEG), axis=0, keepdims=True))
        runsum = runsum + jnp.sum(s * cnt, axis=0, keepdims=True)
    m_meas = runmax - runsum * (1.0 / L)

    iota = jax.lax.broadcasted_iota(jnp.int32, (1, L), 1)

    def body(r, mv):
        mx = jnp.max(mv)
        amin = jnp.min(jnp.where(mv == mx, iota, L))
        idx_scr[r] = amin
        return jnp.where(iota == amin, NEG, mv)

    jax.lax.fori_loop(0, U, body, m_meas)

    rows = [q_scr[pl.ds(idx_scr[r], 1), :] for r in range(U)]
    q_sel = jnp.concatenate(rows, axis=0)
    scores = jax.lax.dot_general(q_sel, k, (((1,), (1,)), ((), ())),
                                 preferred_element_type=jnp.float32)
    smax = jnp.max(scores, axis=1, keepdims=True)
    e = jnp.exp(scores - smax)
    attn = e / jnp.sum(e, axis=1, keepdims=True)
    upd = jnp.dot(attn, v, preferred_element_type=jnp.float32)

    meanv = jnp.mean(v, axis=0, keepdims=True)
    ctx_ref[0] = jnp.broadcast_to(meanv, (L, DK))
    for r in range(U):
        ctx_ref[0, pl.ds(idx_scr[r], 1), :] = upd[r:r + 1, :]


def _out_kernel(ctx_ref, res_ref, wfc_ref, bfc_ref, g_ref, b_ref, o_ref):
    t = jnp.dot(ctx_ref[...], wfc_ref[...], preferred_element_type=jnp.float32)
    t = t + bfc_ref[...] + res_ref[...]
    mu = jnp.mean(t, axis=1, keepdims=True)
    d = t - mu
    var = jnp.mean(d * d, axis=1, keepdims=True)
    o_ref[...] = d * jax.lax.rsqrt(var + EPS) * g_ref[...] + b_ref[...]


def kernel(hidden_states, Wq, Wk, Wv, Wfc, bfc, gamma, beta):
    x = hidden_states.reshape(L, DM)
    cnt_t = jnp.asarray(_CNT_T)
    wq3 = Wq.reshape(DM, H, DK).transpose(1, 0, 2)
    wk3 = Wk.reshape(DM, H, DK).transpose(1, 0, 2)
    wv3 = Wv.reshape(DM, H, DK).transpose(1, 0, 2)

    ctx3 = pl.pallas_call(
        _attn_head_kernel,
        grid=(H,),
        in_specs=[
            pl.BlockSpec((L, DM), lambda h: (0, 0)),
            pl.BlockSpec((1, DM, DK), lambda h: (h, 0, 0)),
            pl.BlockSpec((1, DM, DK), lambda h: (h, 0, 0)),
            pl.BlockSpec((1, DM, DK), lambda h: (h, 0, 0)),
            pl.BlockSpec((L, L), lambda h: (0, 0)),
        ],
        out_specs=pl.BlockSpec((1, L, DK), lambda h: (h, 0, 0)),
        out_shape=jax.ShapeDtypeStruct((H, L, DK), jnp.float32),
        scratch_shapes=[
            pltpu.VMEM((L, DK), jnp.float32),
            pltpu.SMEM((U,), jnp.int32),
        ],
    )(x, wq3, wk3, wv3, cnt_t)
    ctx = ctx3.transpose(1, 0, 2).reshape(L, H * DK)

    BL = 256
    out = pl.pallas_call(
        _out_kernel,
        grid=(L // BL,),
        in_specs=[
            pl.BlockSpec((BL, DM), lambda i: (i, 0)),
            pl.BlockSpec((BL, DM), lambda i: (i, 0)),
            pl.BlockSpec((DM, DM), lambda i: (0, 0)),
            pl.BlockSpec((1, DM), lambda i: (0, 0)),
            pl.BlockSpec((1, DM), lambda i: (0, 0)),
            pl.BlockSpec((1, DM), lambda i: (0, 0)),
        ],
        out_specs=pl.BlockSpec((BL, DM), lambda i: (i, 0)),
        out_shape=jax.ShapeDtypeStruct((L, DM), jnp.float32),
    )(ctx, x, Wfc, bfc.reshape(1, DM), gamma.reshape(1, DM),
      beta.reshape(1, DM))

    return out.reshape(1, L, DM)
